# baseline (speedup 1.0000x reference)
import jax
import jax.numpy as jnp
from jax import lax
from jax.experimental import pallas as pl
from jax.experimental.pallas import tpu as pltpu
from jax.experimental.pallas import tpu_sc as plsc

B = 16
T = 1024
L = 16
NCHUNK = T // L
PAD = L
NEG_INF = float("-inf")


def _sc_body(dist_hbm, frames_hbm, out_hbm, vrow, vf, vout):
    cid = lax.axis_index("c")
    sid = lax.axis_index("s")

    @pl.when(cid == 0)
    def _():
        b = sid
        vrow[pl.ds(0, PAD)] = jnp.full((PAD,), NEG_INF, jnp.float32)
        pltpu.sync_copy(dist_hbm.at[pl.ds(b * 2 * T, 2 * T)],
                        vrow.at[pl.ds(PAD, 2 * T)])
        pltpu.sync_copy(frames_hbm.at[pl.ds(b * T, T)], vf)

        lane = lax.broadcasted_iota(jnp.int32, (L,), 0)
        lane2 = lane * 2

        def phase_a(i, carry):
            rv, bestv, bestc = carry
            s_idx = lane2 + (PAD + 2 * L * i)
            sh_vec = plsc.load_gather(vrow, [s_idx - 2])
            e_vec = plsc.load_gather(vrow, [s_idx + 1])
            s_vec = plsc.load_gather(vrow, [s_idx])
            pfx = jnp.maximum(plsc.cummax(sh_vec), rv)
            cand = pfx + e_vec
            upd = cand > bestv
            bestc = jnp.where(upd, i, bestc)
            bestv = jnp.where(upd, cand, bestv)
            rv = jnp.maximum(rv, jnp.max(s_vec))
            return rv, bestv, bestc

        init_a = (jnp.full((L,), NEG_INF, jnp.float32),
                  jnp.full((L,), NEG_INF, jnp.float32),
                  jnp.zeros((L,), jnp.int32))
        _, bestv, bestc = lax.fori_loop(0, NCHUNK, phase_a, init_a)

        gmax = jnp.max(bestv)
        e_cand = bestc * L + lane
        e_star = jnp.min(jnp.where(bestv == gmax, e_cand, T))
        chunk_e = lax.shift_right_logical(e_star, 4)
        lane_e = jnp.bitwise_and(e_star, L - 1)

        def phase_b(i, carry):
            sbv, sbc = carry
            s_idx = lane2 + (PAD + 2 * L * i)
            s_vec = plsc.load_gather(vrow, [s_idx])
            valid = jnp.logical_or(i < chunk_e, lane < lane_e)
            sv = jnp.where(valid, s_vec, NEG_INF)
            upd = sv > sbv
            sbc = jnp.where(upd, i, sbc)
            sbv = jnp.where(upd, sv, sbv)
            return sbv, sbc

        init_b = (jnp.full((L,), NEG_INF, jnp.float32),
                  jnp.zeros((L,), jnp.int32))
        sbv, sbc = lax.fori_loop(0, chunk_e + 1, phase_b, init_b)
        smax = jnp.max(sbv)
        s_star = jnp.min(jnp.where(sbv == smax, sbc * L + lane, T))

        idx = jnp.where(lane == 0, s_star, e_star)
        frames = plsc.load_gather(vf, [idx])
        res = frames.astype(jnp.float32) + jnp.where(
            lane == 1, jnp.float32(1.0), jnp.float32(0.0))
        vout[...] = res
        pltpu.sync_copy(vout, out_hbm.at[pl.ds(b * L, L)])


@jax.jit
def _post_process(dist, frames):
    mesh = plsc.VectorSubcoreMesh(core_axis_name="c", subcore_axis_name="s")
    run = pl.kernel(
        _sc_body,
        out_type=jax.ShapeDtypeStruct((B * L,), jnp.float32),
        mesh=mesh,
        compiler_params=pltpu.CompilerParams(needs_layout_passes=False),
        scratch_types=[
            pltpu.VMEM((PAD + 2 * T,), jnp.float32),
            pltpu.VMEM((T,), jnp.int32),
            pltpu.VMEM((L,), jnp.float32),
        ],
    )
    out = run(dist, frames)
    return out.reshape(B, L)[:, :2]


def kernel(temporal_dist, time_mask, frames_id):
    del time_mask
    dist = temporal_dist.reshape(-1)
    frames = frames_id.reshape(-1).astype(jnp.int32)
    return _post_process(dist, frames)


# speedup vs baseline: 1.4344x; 1.4344x over previous
import jax
import jax.numpy as jnp
from jax import lax
from jax.experimental import pallas as pl
from jax.experimental.pallas import tpu as pltpu
from jax.experimental.pallas import tpu_sc as plsc

B = 16
T = 1024
L = 16
NCHUNK = T // L
PAD = L
NEG_INF = float("-inf")


def _sc_body(starts_hbm, ends_hbm, frames_hbm, out_hbm, vs, ve, vf, vout):
    cid = lax.axis_index("c")
    sid = lax.axis_index("s")

    @pl.when(cid == 0)
    def _():
        b = sid
        vs[pl.ds(0, PAD)] = jnp.full((PAD,), NEG_INF, jnp.float32)
        pltpu.sync_copy(starts_hbm.at[b], vs.at[pl.ds(PAD, T)])
        pltpu.sync_copy(ends_hbm.at[b], ve)
        pltpu.sync_copy(frames_hbm.at[b], vf)

        lane = lax.broadcasted_iota(jnp.int32, (L,), 0)

        def phase_a(i, carry):
            rv, bestv, bestc = carry
            base = PAD + i * L
            s_vec = vs[pl.ds(base, L)]
            sh_vec = vs[pl.ds(base - 1, L)]
            e_vec = ve[pl.ds(i * L, L)]
            pfx = jnp.maximum(plsc.cummax(sh_vec), rv)
            cand = pfx + e_vec
            upd = cand > bestv
            bestc = jnp.where(upd, i, bestc)
            bestv = jnp.where(upd, cand, bestv)
            rv = jnp.maximum(rv, jnp.max(s_vec))
            return rv, bestv, bestc

        init_a = (jnp.full((L,), NEG_INF, jnp.float32),
                  jnp.full((L,), NEG_INF, jnp.float32),
                  jnp.zeros((L,), jnp.int32))
        _, bestv, bestc = lax.fori_loop(0, NCHUNK, phase_a, init_a)

        gmax = jnp.max(bestv)
        e_cand = bestc * L + lane
        e_star = jnp.min(jnp.where(bestv == gmax, e_cand, T))
        chunk_e = lax.shift_right_logical(e_star, 4)
        lane_e = jnp.bitwise_and(e_star, L - 1)

        def phase_b(i, carry):
            sbv, sbc = carry
            s_vec = vs[pl.ds(PAD + i * L, L)]
            valid = jnp.logical_or(i < chunk_e, lane < lane_e)
            sv = jnp.where(valid, s_vec, NEG_INF)
            upd = sv > sbv
            sbc = jnp.where(upd, i, sbc)
            sbv = jnp.where(upd, sv, sbv)
            return sbv, sbc

        init_b = (jnp.full((L,), NEG_INF, jnp.float32),
                  jnp.zeros((L,), jnp.int32))
        sbv, sbc = lax.fori_loop(0, chunk_e + 1, phase_b, init_b)
        smax = jnp.max(sbv)
        s_star = jnp.min(jnp.where(sbv == smax, sbc * L + lane, T))

        idx = jnp.where(lane == 0, s_star, e_star)
        frames = plsc.load_gather(vf, [idx])
        res = frames.astype(jnp.float32) + jnp.where(
            lane == 1, jnp.float32(1.0), jnp.float32(0.0))
        vout[...] = res
        pltpu.sync_copy(vout, out_hbm.at[b])


@jax.jit
def _post_process(starts, ends, frames):
    mesh = plsc.VectorSubcoreMesh(core_axis_name="c", subcore_axis_name="s")
    run = pl.kernel(
        _sc_body,
        out_type=jax.ShapeDtypeStruct((B, L), jnp.float32),
        mesh=mesh,
        compiler_params=pltpu.CompilerParams(
            needs_layout_passes=False, use_tc_tiling_on_sc=False),
        scratch_types=[
            pltpu.VMEM((PAD + T,), jnp.float32),
            pltpu.VMEM((T,), jnp.float32),
            pltpu.VMEM((T,), jnp.int32),
            pltpu.VMEM((L,), jnp.float32),
        ],
    )
    out = run(starts, ends, frames)
    return out[:, :2]


def kernel(temporal_dist, time_mask, frames_id):
    del time_mask
    starts = temporal_dist[:, :, 0]
    ends = temporal_dist[:, :, 1]
    frames = frames_id.astype(jnp.int32)
    return _post_process(starts, ends, frames)


# speedup vs baseline: 1.5169x; 1.0575x over previous
import jax
import jax.numpy as jnp
from jax import lax
from jax.experimental import pallas as pl
from jax.experimental.pallas import tpu as pltpu
from jax.experimental.pallas import tpu_sc as plsc

B = 16
T = 1024
L = 16
NCHUNK = T // L
PAD = L
NEG_INF = float("-inf")


def _sc_body(starts_hbm, ends_hbm, frames_hbm, out_hbm, vs, ve, vf, vout):
    cid = lax.axis_index("c")
    sid = lax.axis_index("s")

    @pl.when(cid == 0)
    def _():
        b = sid
        vs[pl.ds(0, PAD)] = jnp.full((PAD,), NEG_INF, jnp.float32)
        pltpu.sync_copy(starts_hbm.at[b], vs.at[pl.ds(PAD, T)])
        pltpu.sync_copy(ends_hbm.at[b], ve)
        pltpu.sync_copy(frames_hbm.at[b], vf)

        lane = lax.broadcasted_iota(jnp.int32, (L,), 0)

        def phase_a(i, carry):
            rv, bestv, bestc = carry
            base = PAD + i * L
            s_vec = vs[pl.ds(base, L)]
            sh_vec = vs[pl.ds(base - 1, L)]
            e_vec = ve[pl.ds(i * L, L)]
            pfx = jnp.maximum(plsc.cummax(sh_vec), rv)
            cand = pfx + e_vec
            upd = cand > bestv
            bestc = jnp.where(upd, i, bestc)
            bestv = jnp.where(upd, cand, bestv)
            rv = jnp.maximum(rv, jnp.max(s_vec))
            return rv, bestv, bestc

        init_a = (jnp.full((L,), NEG_INF, jnp.float32),
                  jnp.full((L,), NEG_INF, jnp.float32),
                  jnp.zeros((L,), jnp.int32))
        _, bestv, bestc = lax.fori_loop(0, NCHUNK, phase_a, init_a)

        gmax = jnp.max(bestv)
        e_cand = bestc * L + lane
        e_star = jnp.min(jnp.where(bestv == gmax, e_cand, T))
        chunk_e = lax.shift_right_logical(e_star, 4)
        lane_e = jnp.bitwise_and(e_star, L - 1)

        def phase_b(i, carry):
            sbv, sbc = carry
            s_vec = vs[pl.ds(PAD + i * L, L)]
            valid = jnp.logical_or(i < chunk_e, lane < lane_e)
            sv = jnp.where(valid, s_vec, NEG_INF)
            upd = sv > sbv
            sbc = jnp.where(upd, i, sbc)
            sbv = jnp.where(upd, sv, sbv)
            return sbv, sbc

        init_b = (jnp.full((L,), NEG_INF, jnp.float32),
                  jnp.zeros((L,), jnp.int32))
        sbv, sbc = lax.fori_loop(0, chunk_e + 1, phase_b, init_b)
        smax = jnp.max(sbv)
        s_star = jnp.min(jnp.where(sbv == smax, sbc * L + lane, T))

        idx = jnp.where(lane == 0, s_star, e_star)
        frames = plsc.load_gather(vf, [idx])
        res = frames.astype(jnp.float32) + jnp.where(
            lane == 1, jnp.float32(1.0), jnp.float32(0.0))
        vout[...] = res
        pltpu.sync_copy(vout, out_hbm.at[b])


@jax.jit
def _post_process(starts, ends, frames):
    mesh = plsc.VectorSubcoreMesh(
        core_axis_name="c", subcore_axis_name="s", num_cores=1)
    run = pl.kernel(
        _sc_body,
        out_type=jax.ShapeDtypeStruct((B, L), jnp.float32),
        mesh=mesh,
        compiler_params=pltpu.CompilerParams(
            needs_layout_passes=False, use_tc_tiling_on_sc=False),
        scratch_types=[
            pltpu.VMEM((PAD + T,), jnp.float32),
            pltpu.VMEM((T,), jnp.float32),
            pltpu.VMEM((T,), jnp.int32),
            pltpu.VMEM((L,), jnp.float32),
        ],
    )
    out = run(starts, ends, frames)
    return out[:, :2]


def kernel(temporal_dist, time_mask, frames_id):
    del time_mask
    starts = temporal_dist[:, :, 0]
    ends = temporal_dist[:, :, 1]
    frames = frames_id.astype(jnp.int32)
    return _post_process(starts, ends, frames)


# speedup vs baseline: 1.5437x; 1.0177x over previous
import jax
import jax.numpy as jnp
from jax import lax
from jax.experimental import pallas as pl
from jax.experimental.pallas import tpu as pltpu
from jax.experimental.pallas import tpu_sc as plsc

B = 16
T = 1024
L = 16
NCHUNK = T // L
NEG_INF = float("-inf")


def _sc_body(packed_hbm, out_hbm, vs, ve, vf, vout):
    b = lax.axis_index("s")
    pltpu.sync_copy(packed_hbm.at[b, 0], vs)
    pltpu.sync_copy(packed_hbm.at[b, 1], ve)
    pltpu.sync_copy(packed_hbm.at[b, 2], vf)

    lane = lax.broadcasted_iota(jnp.int32, (L,), 0)
    lshift = jnp.maximum(lane - 1, 0)

    def load16(ref, i):
        return ref[lax.shift_right_logical(i, 3),
                   pl.ds(jnp.bitwise_and(i, 7) * L, L)]

    def phase_a(i, carry):
        rv, bestv, bestc = carry
        s_vec = load16(vs, i)
        e_vec = load16(ve, i)
        incl = plsc.cummax(s_vec)
        ex = incl.at[lshift].get(mode="promise_in_bounds")
        pfx = jnp.maximum(jnp.where(lane == 0, NEG_INF, ex), rv)
        cand = pfx + e_vec
        upd = cand > bestv
        bestc = jnp.where(upd, i, bestc)
        bestv = jnp.where(upd, cand, bestv)
        rv = jnp.maximum(rv, jnp.max(s_vec))
        return rv, bestv, bestc

    init_a = (jnp.full((L,), NEG_INF, jnp.float32),
              jnp.full((L,), NEG_INF, jnp.float32),
              jnp.zeros((L,), jnp.int32))
    _, bestv, bestc = lax.fori_loop(0, NCHUNK, phase_a, init_a)

    gmax = jnp.max(bestv)
    e_cand = bestc * L + lane
    e_star = jnp.min(jnp.where(bestv == gmax, e_cand, T))
    chunk_e = lax.shift_right_logical(e_star, 4)
    lane_e = jnp.bitwise_and(e_star, L - 1)

    def phase_b(i, carry):
        sbv, sbc = carry
        s_vec = load16(vs, i)
        valid = jnp.logical_or(i < chunk_e, lane < lane_e)
        sv = jnp.where(valid, s_vec, NEG_INF)
        upd = sv > sbv
        sbc = jnp.where(upd, i, sbc)
        sbv = jnp.where(upd, sv, sbv)
        return sbv, sbc

    init_b = (jnp.full((L,), NEG_INF, jnp.float32),
              jnp.zeros((L,), jnp.int32))
    sbv, sbc = lax.fori_loop(0, chunk_e + 1, phase_b, init_b)
    smax = jnp.max(sbv)
    s_star = jnp.min(jnp.where(sbv == smax, sbc * L + lane, T))

    idx = jnp.where(lane == 0, s_star, e_star)
    frames = plsc.bitcast(
        plsc.load_gather(vf, [lax.shift_right_logical(idx, 7),
                              jnp.bitwise_and(idx, 127)]),
        jnp.int32)
    res = frames.astype(jnp.float32) + jnp.where(
        lane == 1, jnp.float32(1.0), jnp.float32(0.0))
    vout[...] = res
    pltpu.sync_copy(vout, out_hbm.at[b, 0, pl.ds(0, L)])


@jax.jit
def _post_process(packed):
    mesh = plsc.VectorSubcoreMesh(
        core_axis_name="c", subcore_axis_name="s", num_cores=1)
    run = pl.kernel(
        _sc_body,
        out_type=jax.ShapeDtypeStruct((B, 8, 128), jnp.float32),
        mesh=mesh,
        compiler_params=pltpu.CompilerParams(needs_layout_passes=False),
        scratch_types=[
            pltpu.VMEM((8, 128), jnp.float32),
            pltpu.VMEM((8, 128), jnp.float32),
            pltpu.VMEM((8, 128), jnp.float32),
            pltpu.VMEM((L,), jnp.float32),
        ],
    )
    out = run(packed)
    return out[:, 0, :2]


def kernel(temporal_dist, time_mask, frames_id):
    del time_mask
    frames_f = lax.bitcast_convert_type(frames_id.astype(jnp.int32),
                                        jnp.float32)
    packed = jnp.stack(
        [temporal_dist[:, :, 0], temporal_dist[:, :, 1], frames_f],
        axis=1).reshape(B, 3, 8, 128)
    return _post_process(packed)


# speedup vs baseline: 1.5498x; 1.0040x over previous
import jax
import jax.numpy as jnp
from jax import lax
from jax.experimental import pallas as pl
from jax.experimental.pallas import tpu as pltpu
from jax.experimental.pallas import tpu_sc as plsc

B = 16
T = 1024
L = 16
NCHUNK = T // L
NEG_INF = float("-inf")


def _sc_body(packed_hbm, out_hbm, vs, ve, vf, vout):
    b = lax.axis_index("s")
    pltpu.sync_copy(packed_hbm.at[b, 0], vs)
    pltpu.sync_copy(packed_hbm.at[b, 1], ve)
    pltpu.sync_copy(packed_hbm.at[b, 2], vf)

    lane = lax.broadcasted_iota(jnp.int32, (L,), 0)
    lshift = jnp.maximum(lane - 1, 0)

    def load16(ref, i):
        return ref[lax.shift_right_logical(i, 3),
                   pl.ds(jnp.bitwise_and(i, 7) * L, L)]

    def phase_a(i, carry):
        rv, bestv, bestc = carry
        s_vec = load16(vs, i)
        e_vec = load16(ve, i)
        incl = plsc.cummax(s_vec)
        ex = incl.at[lshift].get(mode="promise_in_bounds")
        pfx = jnp.maximum(jnp.where(lane == 0, NEG_INF, ex), rv)
        cand = pfx + e_vec
        upd = cand > bestv
        bestc = jnp.where(upd, i, bestc)
        bestv = jnp.where(upd, cand, bestv)
        rv = jnp.maximum(rv, jnp.max(s_vec))
        return rv, bestv, bestc

    init_a = (jnp.full((L,), NEG_INF, jnp.float32),
              jnp.full((L,), NEG_INF, jnp.float32),
              jnp.zeros((L,), jnp.int32))
    _, bestv, bestc = lax.fori_loop(0, NCHUNK, phase_a, init_a)

    gmax = jnp.max(bestv)
    e_cand = bestc * L + lane
    e_star = jnp.min(jnp.where(bestv == gmax, e_cand, T))
    chunk_e = lax.shift_right_logical(e_star, 4)
    lane_e = jnp.bitwise_and(e_star, L - 1)

    def phase_b(i, carry):
        sbv, sbc = carry
        s_vec = load16(vs, i)
        valid = jnp.logical_or(i < chunk_e, lane < lane_e)
        sv = jnp.where(valid, s_vec, NEG_INF)
        upd = sv > sbv
        sbc = jnp.where(upd, i, sbc)
        sbv = jnp.where(upd, sv, sbv)
        return sbv, sbc

    init_b = (jnp.full((L,), NEG_INF, jnp.float32),
              jnp.zeros((L,), jnp.int32))
    sbv, sbc = lax.fori_loop(0, chunk_e + 1, phase_b, init_b)
    smax = jnp.max(sbv)
    s_star = jnp.min(jnp.where(sbv == smax, sbc * L + lane, T))

    idx = jnp.where(lane == 0, s_star, e_star)
    frames = plsc.bitcast(
        plsc.load_gather(vf, [lax.shift_right_logical(idx, 7),
                              jnp.bitwise_and(idx, 127)]),
        jnp.int32)
    res = frames.astype(jnp.float32) + jnp.where(
        lane == 1, jnp.float32(1.0), jnp.float32(0.0))
    vout[...] = res
    pltpu.sync_copy(vout, out_hbm.at[b, 0, pl.ds(0, L)])


@jax.jit
def _post_process(packed):
    mesh = plsc.VectorSubcoreMesh(
        core_axis_name="c", subcore_axis_name="s", num_cores=1)
    run = pl.kernel(
        _sc_body,
        out_type=jax.ShapeDtypeStruct((B, 8, 128), jnp.float32),
        mesh=mesh,
        compiler_params=pltpu.CompilerParams(
            needs_layout_passes=False, skip_device_barrier=True),
        scratch_types=[
            pltpu.VMEM((8, 128), jnp.float32),
            pltpu.VMEM((8, 128), jnp.float32),
            pltpu.VMEM((8, 128), jnp.float32),
            pltpu.VMEM((L,), jnp.float32),
        ],
    )
    out = run(packed)
    return out[:, 0, :2]


def kernel(temporal_dist, time_mask, frames_id):
    del time_mask
    frames_f = lax.bitcast_convert_type(frames_id.astype(jnp.int32),
                                        jnp.float32)
    packed = jnp.stack(
        [temporal_dist[:, :, 0], temporal_dist[:, :, 1], frames_f],
        axis=1).reshape(B, 3, 8, 128)
    return _post_process(packed)


# speedup vs baseline: 1.5647x; 1.0096x over previous
import jax
import jax.numpy as jnp
from jax import lax
from jax.experimental import pallas as pl
from jax.experimental.pallas import tpu as pltpu
from jax.experimental.pallas import tpu_sc as plsc

B = 16
T = 1024
L = 16
NCHUNK = T // L
NEG_INF = float("-inf")


def _sc_body(starts_hbm, ends_hbm, frames_hbm, out_hbm, vs, ve, vf, vout):
    b = lax.axis_index("s")
    pltpu.sync_copy(starts_hbm.at[b], vs)
    pltpu.sync_copy(ends_hbm.at[b], ve)
    pltpu.sync_copy(frames_hbm.at[b], vf)

    lane = lax.broadcasted_iota(jnp.int32, (L,), 0)
    lshift = jnp.maximum(lane - 1, 0)

    def load16(ref, i):
        return ref[lax.shift_right_logical(i, 3),
                   pl.ds(jnp.bitwise_and(i, 7) * L, L)]

    def phase_a(i, carry):
        rv, bestv, bestc = carry
        s_vec = load16(vs, i)
        e_vec = load16(ve, i)
        incl = plsc.cummax(s_vec)
        ex = incl.at[lshift].get(mode="promise_in_bounds")
        pfx = jnp.maximum(jnp.where(lane == 0, NEG_INF, ex), rv)
        cand = pfx + e_vec
        upd = cand > bestv
        bestc = jnp.where(upd, i, bestc)
        bestv = jnp.where(upd, cand, bestv)
        rv = jnp.maximum(rv, jnp.max(s_vec))
        return rv, bestv, bestc

    init_a = (jnp.full((L,), NEG_INF, jnp.float32),
              jnp.full((L,), NEG_INF, jnp.float32),
              jnp.zeros((L,), jnp.int32))
    _, bestv, bestc = lax.fori_loop(0, NCHUNK, phase_a, init_a)

    gmax = jnp.max(bestv)
    e_cand = bestc * L + lane
    e_star = jnp.min(jnp.where(bestv == gmax, e_cand, T))
    chunk_e = lax.shift_right_logical(e_star, 4)
    lane_e = jnp.bitwise_and(e_star, L - 1)

    def phase_b(i, carry):
        sbv, sbc = carry
        s_vec = load16(vs, i)
        valid = jnp.logical_or(i < chunk_e, lane < lane_e)
        sv = jnp.where(valid, s_vec, NEG_INF)
        upd = sv > sbv
        sbc = jnp.where(upd, i, sbc)
        sbv = jnp.where(upd, sv, sbv)
        return sbv, sbc

    init_b = (jnp.full((L,), NEG_INF, jnp.float32),
              jnp.zeros((L,), jnp.int32))
    sbv, sbc = lax.fori_loop(0, chunk_e + 1, phase_b, init_b)
    smax = jnp.max(sbv)
    s_star = jnp.min(jnp.where(sbv == smax, sbc * L + lane, T))

    idx = jnp.where(lane == 0, s_star, e_star)
    frames = plsc.load_gather(vf, [lax.shift_right_logical(idx, 7),
                                   jnp.bitwise_and(idx, 127)])
    res = frames.astype(jnp.float32) + jnp.where(
        lane == 1, jnp.float32(1.0), jnp.float32(0.0))
    vout[...] = res
    pltpu.sync_copy(vout, out_hbm.at[b, 0, pl.ds(0, L)])


@jax.jit
def _post_process(starts, ends, frames):
    mesh = plsc.VectorSubcoreMesh(
        core_axis_name="c", subcore_axis_name="s", num_cores=1)
    run = pl.kernel(
        _sc_body,
        out_type=jax.ShapeDtypeStruct((B, 8, 128), jnp.float32),
        mesh=mesh,
        compiler_params=pltpu.CompilerParams(needs_layout_passes=False),
        scratch_types=[
            pltpu.VMEM((8, 128), jnp.float32),
            pltpu.VMEM((8, 128), jnp.float32),
            pltpu.VMEM((8, 128), jnp.int32),
            pltpu.VMEM((L,), jnp.float32),
        ],
    )
    out = run(starts, ends, frames)
    return out[:, 0, :2]


def kernel(temporal_dist, time_mask, frames_id):
    del time_mask
    starts = temporal_dist[:, :, 0].reshape(B, 8, 128)
    ends = temporal_dist[:, :, 1].reshape(B, 8, 128)
    frames = frames_id.astype(jnp.int32).reshape(B, 8, 128)
    return _post_process(starts, ends, frames)
